# baseline (speedup 1.0000x reference)
import functools

import jax
import jax.numpy as jnp
from jax import lax
from jax.experimental import pallas as pl
from jax.experimental.pallas import tpu as pltpu
from jax.experimental.pallas import tpu_sc as plsc

VOCAB = 49408
D = 512
N_CLS = 48
N_CTX = 8
MAX_LEN = 40
SAMPLE_RATE = 4
B = 8
T = 1024 // SAMPLE_RATE
P = 15
L_LAB = 5
SOS_ID = VOCAB - 2
EOS_ID = VOCAB - 1

NPAIR = B * T
NW = 32
PAIRS_PER_W = NPAIR // NW
NCHUNK = PAIRS_PER_W // 2
TAIL = 24
LPT = N_CLS // 16


def _sc_body(tok_emb, ctx_flat, tok16f, labp_h, metac, zrows,
             out, mask, labw,
             abuf, wbuf, mmall, pidx, labp, mrow, gsem, ssem):
    c = lax.axis_index("c")
    s = lax.axis_index("s")
    wid = s * 2 + c
    wbase = wid * PAIRS_PER_W

    zero16 = jnp.zeros((16,), jnp.float32)
    pltpu.sync_copy(metac, mrow)
    for k in range(LPT):
        lab = s * LPT + k
        row = c * N_CLS + lab
        pltpu.sync_copy(ctx_flat.at[pl.ds(lab * N_CTX, N_CTX)],
                        abuf.at[pl.ds(0, 8)])
        pltpu.async_copy(tok_emb.at[mrow.at[lab, pl.ds(0, 8)]],
                         abuf.at[pl.ds(8, 8)], gsem).wait()
        for r in (14, 15):
            for cc in range(D // 16):
                abuf[r, pl.ds(cc * 16, 16)] = zero16
        pltpu.sync_copy(zrows, abuf.at[pl.ds(16, 8)])
        pltpu.sync_copy(abuf.at[pl.ds(0, TAIL)], labw.at[row])
    plsc.subcore_barrier()

    pltpu.sync_copy(tok16f.at[pl.ds(wbase * 16, PAIRS_PER_W * 16)], pidx)
    pltpu.sync_copy(labp_h.at[pl.ds(wid * NCHUNK, NCHUNK)], labp)

    lanes = lax.iota(jnp.int32, 16)
    one16 = jnp.ones((16,), jnp.float32)

    def body(ch, carry):
        @pl.when(ch > 0)
        def _():
            for k in range(2):
                pltpu.make_async_copy(abuf.at[pl.ds(0, 16)],
                                      out.at[pl.ds(0, 16)], ssem).wait()
                pltpu.make_async_copy(wbuf.at[k],
                                      out.at[pl.ds(0, TAIL)], ssem).wait()

        da = pltpu.async_copy(tok_emb.at[pidx.at[pl.ds(ch * 32, 32)]],
                              abuf, gsem)
        dw = pltpu.async_copy(labw.at[labp.at[ch, pl.ds(0, 2)]], wbuf, gsem)
        da.wait()
        dw.wait()

        for k in range(2):
            acc0 = zero16
            for r in range(16):
                x = abuf[k * 16 + r, pl.ds(0, 16)]
                acc0 = jnp.where(lanes == r,
                                 jnp.broadcast_to(x[0:1], (16,)), acc0)
            acc1 = zero16
            for r in range(14):
                x = wbuf[k, r, pl.ds(0, 16)]
                acc1 = jnp.where(lanes == r,
                                 jnp.broadcast_to(x[0:1], (16,)), acc1)
            poff = (ch * 2 + k) * MAX_LEN
            mmall[pl.ds(poff, 16)] = jnp.where(acc0 != 0.0, one16, zero16)
            mmall[pl.ds(poff + 16, 16)] = jnp.where(acc1 != 0.0, one16, zero16)
            mmall[pl.ds(poff + 32, 16)] = zero16
            n = wbase + ch * 2 + k
            pltpu.async_copy(abuf.at[pl.ds(k * 16, 16)],
                             out.at[pl.ds(n * MAX_LEN, 16)], ssem)
            pltpu.async_copy(wbuf.at[k],
                             out.at[pl.ds(n * MAX_LEN + 16, TAIL)], ssem)
        return carry

    lax.fori_loop(0, NCHUNK, body, 0)

    for k in range(2):
        pltpu.make_async_copy(abuf.at[pl.ds(0, 16)],
                              out.at[pl.ds(0, 16)], ssem).wait()
        pltpu.make_async_copy(wbuf.at[k],
                              out.at[pl.ds(0, TAIL)], ssem).wait()
    pltpu.sync_copy(mmall.at[pl.ds(0, PAIRS_PER_W * MAX_LEN)],
                    mask.at[pl.ds(wbase * MAX_LEN, PAIRS_PER_W * MAX_LEN)])


def kernel(token_embedding, ctx, last_clip_labels, prompt_prefix_tokens, label_tokens):
    labels_s = last_clip_labels[:, ::SAMPLE_RATE].reshape(NPAIR).astype(jnp.int32)
    tok16f = jnp.concatenate(
        [jnp.full((NPAIR, 1), SOS_ID, jnp.int32),
         prompt_prefix_tokens.reshape(NPAIR, P).astype(jnp.int32)],
        axis=1).reshape(NPAIR * 16)
    ctx_flat = ctx.reshape(N_CLS * N_CTX, D)
    sc_of_pair = (jnp.arange(NPAIR, dtype=jnp.int32) // PAIRS_PER_W) % 2
    labp_h = jnp.pad((labels_s + N_CLS * sc_of_pair).reshape(NPAIR // 2, 2),
                     ((0, 0), (0, 6)))
    metac = jnp.concatenate(
        [label_tokens.astype(jnp.int32),
         jnp.full((N_CLS, 128 - L_LAB), EOS_ID, jnp.int32)], axis=1)
    zrows = jnp.zeros((8, D), jnp.float32)

    mesh = plsc.VectorSubcoreMesh(core_axis_name="c", subcore_axis_name="s")
    run = functools.partial(
        pl.kernel,
        out_type=(jax.ShapeDtypeStruct((NPAIR * MAX_LEN, D), jnp.float32),
                  jax.ShapeDtypeStruct((NPAIR * MAX_LEN,), jnp.float32),
                  jax.ShapeDtypeStruct((2 * N_CLS, TAIL, D), jnp.float32)),
        mesh=mesh,
        scratch_types=[
            pltpu.VMEM((32, D), jnp.float32),
            pltpu.VMEM((2, TAIL, D), jnp.float32),
            pltpu.VMEM((PAIRS_PER_W * MAX_LEN + 8,), jnp.float32),
            pltpu.VMEM((PAIRS_PER_W * 16,), jnp.int32),
            pltpu.VMEM((NCHUNK, 8), jnp.int32),
            pltpu.VMEM((N_CLS, 128), jnp.int32),
            pltpu.SemaphoreType.DMA,
            pltpu.SemaphoreType.DMA,
        ],
    )(_sc_body)
    out, mask, _ = run(token_embedding, ctx_flat, tok16f, labp_h, metac, zrows)
    prompts = out.reshape(B, T, MAX_LEN, D)
    pad_masks = mask.reshape(B, T, MAX_LEN, 1)
    return (prompts, pad_masks)


# speedup vs baseline: 1.0227x; 1.0227x over previous
import functools

import jax
import jax.numpy as jnp
from jax import lax
from jax.experimental import pallas as pl
from jax.experimental.pallas import tpu as pltpu
from jax.experimental.pallas import tpu_sc as plsc

VOCAB = 49408
D = 512
N_CLS = 48
N_CTX = 8
MAX_LEN = 40
SAMPLE_RATE = 4
B = 8
T = 1024 // SAMPLE_RATE
P = 15
L_LAB = 5
SOS_ID = VOCAB - 2
EOS_ID = VOCAB - 1

NPAIR = B * T
NW = 32
PAIRS_PER_W = NPAIR // NW
CPAIRS = 4
NCHUNK = PAIRS_PER_W // CPAIRS
TAIL = 24
LPT = N_CLS // 16


def _sc_body(tok_emb, ctx_flat, tok16f, labp_h, metac, zrows,
             out, mask, labw,
             abuf, wbuf, mmall, pidx, labp, mrow, gsem, ssem):
    c = lax.axis_index("c")
    s = lax.axis_index("s")
    wid = s * 2 + c
    wbase = wid * PAIRS_PER_W

    zero16 = jnp.zeros((16,), jnp.float32)
    pltpu.sync_copy(metac, mrow)
    for k in range(LPT):
        lab = s * LPT + k
        row = c * N_CLS + lab
        pltpu.sync_copy(ctx_flat.at[pl.ds(lab * N_CTX, N_CTX)],
                        abuf.at[pl.ds(0, 8)])
        pltpu.async_copy(tok_emb.at[mrow.at[lab, pl.ds(0, 8)]],
                         abuf.at[pl.ds(8, 8)], gsem).wait()
        for r in (14, 15):
            for cc in range(D // 16):
                abuf[r, pl.ds(cc * 16, 16)] = zero16
        pltpu.sync_copy(zrows, abuf.at[pl.ds(16, 8)])
        pltpu.sync_copy(abuf.at[pl.ds(0, TAIL)], labw.at[row])
    plsc.subcore_barrier()

    pltpu.sync_copy(tok16f.at[pl.ds(wbase * 16, PAIRS_PER_W * 16)], pidx)
    pltpu.sync_copy(labp_h.at[pl.ds(wid * NCHUNK, NCHUNK)], labp)

    lanes = lax.iota(jnp.int32, 16)
    one16 = jnp.ones((16,), jnp.float32)

    def body(ch, carry):
        @pl.when(ch > 0)
        def _():
            for k in range(CPAIRS):
                pltpu.make_async_copy(abuf.at[pl.ds(0, 16)],
                                      out.at[pl.ds(0, 16)], ssem).wait()
                pltpu.make_async_copy(wbuf.at[k],
                                      out.at[pl.ds(0, TAIL)], ssem).wait()

        da = pltpu.async_copy(tok_emb.at[pidx.at[pl.ds(ch * (16 * CPAIRS), 16 * CPAIRS)]],
                              abuf, gsem)
        dw = pltpu.async_copy(labw.at[labp.at[ch, pl.ds(0, CPAIRS)]], wbuf, gsem)
        da.wait()
        dw.wait()

        for k in range(CPAIRS):
            acc0 = zero16
            for r in range(16):
                x = abuf[k * 16 + r, pl.ds(0, 16)]
                acc0 = jnp.where(lanes == r,
                                 jnp.broadcast_to(x[0:1], (16,)), acc0)
            acc1 = zero16
            for r in range(14):
                x = wbuf[k, r, pl.ds(0, 16)]
                acc1 = jnp.where(lanes == r,
                                 jnp.broadcast_to(x[0:1], (16,)), acc1)
            poff = (ch * CPAIRS + k) * MAX_LEN
            mmall[pl.ds(poff, 16)] = jnp.where(acc0 != 0.0, one16, zero16)
            mmall[pl.ds(poff + 16, 16)] = jnp.where(acc1 != 0.0, one16, zero16)
            mmall[pl.ds(poff + 32, 16)] = zero16
            n = wbase + ch * CPAIRS + k
            pltpu.async_copy(abuf.at[pl.ds(k * 16, 16)],
                             out.at[pl.ds(n * MAX_LEN, 16)], ssem)
            pltpu.async_copy(wbuf.at[k],
                             out.at[pl.ds(n * MAX_LEN + 16, TAIL)], ssem)
        return carry

    lax.fori_loop(0, NCHUNK, body, 0)

    for k in range(CPAIRS):
        pltpu.make_async_copy(abuf.at[pl.ds(0, 16)],
                              out.at[pl.ds(0, 16)], ssem).wait()
        pltpu.make_async_copy(wbuf.at[k],
                              out.at[pl.ds(0, TAIL)], ssem).wait()
    pltpu.sync_copy(mmall.at[pl.ds(0, PAIRS_PER_W * MAX_LEN)],
                    mask.at[pl.ds(wbase * MAX_LEN, PAIRS_PER_W * MAX_LEN)])


def kernel(token_embedding, ctx, last_clip_labels, prompt_prefix_tokens, label_tokens):
    labels_s = last_clip_labels[:, ::SAMPLE_RATE].reshape(NPAIR).astype(jnp.int32)
    tok16f = jnp.concatenate(
        [jnp.full((NPAIR, 1), SOS_ID, jnp.int32),
         prompt_prefix_tokens.reshape(NPAIR, P).astype(jnp.int32)],
        axis=1).reshape(NPAIR * 16)
    ctx_flat = ctx.reshape(N_CLS * N_CTX, D)
    sc_of_pair = (jnp.arange(NPAIR, dtype=jnp.int32) // PAIRS_PER_W) % 2
    labp_h = jnp.pad(
        (labels_s + N_CLS * sc_of_pair).reshape(NPAIR // CPAIRS, CPAIRS),
        ((0, 0), (0, 8 - CPAIRS)))
    metac = jnp.concatenate(
        [label_tokens.astype(jnp.int32),
         jnp.full((N_CLS, 128 - L_LAB), EOS_ID, jnp.int32)], axis=1)
    zrows = jnp.zeros((8, D), jnp.float32)

    mesh = plsc.VectorSubcoreMesh(core_axis_name="c", subcore_axis_name="s")
    run = functools.partial(
        pl.kernel,
        out_type=(jax.ShapeDtypeStruct((NPAIR * MAX_LEN, D), jnp.float32),
                  jax.ShapeDtypeStruct((NPAIR * MAX_LEN,), jnp.float32),
                  jax.ShapeDtypeStruct((2 * N_CLS, TAIL, D), jnp.float32)),
        mesh=mesh,
        scratch_types=[
            pltpu.VMEM((16 * CPAIRS, D), jnp.float32),
            pltpu.VMEM((CPAIRS, TAIL, D), jnp.float32),
            pltpu.VMEM((PAIRS_PER_W * MAX_LEN + 8,), jnp.float32),
            pltpu.VMEM((PAIRS_PER_W * 16,), jnp.int32),
            pltpu.VMEM((NCHUNK, 8), jnp.int32),
            pltpu.VMEM((N_CLS, 128), jnp.int32),
            pltpu.SemaphoreType.DMA,
            pltpu.SemaphoreType.DMA,
        ],
    )(_sc_body)
    out, mask, _ = run(token_embedding, ctx_flat, tok16f, labp_h, metac, zrows)
    prompts = out.reshape(B, T, MAX_LEN, D)
    pad_masks = mask.reshape(B, T, MAX_LEN, 1)
    return (prompts, pad_masks)
